# baseline (speedup 1.0000x reference)
import functools

import jax
import jax.numpy as jnp
from jax import lax
from jax.experimental import pallas as pl
from jax.experimental.pallas import tpu as pltpu
from jax.experimental.pallas import tpu_sc as plsc

NC = 2
NS = 16
NW = NC * NS
LANES = 16
CHUNK = 128


def _make_kernel(B, S, V, D):
    rows_total = B * S
    rows_per_w = rows_total // NW
    chunks = rows_per_w // CHUNK
    assert rows_per_w % CHUNK == 0 and rows_per_w % S == 0
    assert D == 64 and V % 2 == 0

    mesh = plsc.VectorSubcoreMesh(
        core_axis_name="c", subcore_axis_name="s",
        num_cores=NC, num_subcores=NS)

    @functools.partial(
        pl.kernel,
        out_type=jax.ShapeDtypeStruct((rows_total // 2, 2 * D), jnp.float32),
        mesh=mesh,
        scratch_types=[
            pltpu.VMEM((CHUNK,), jnp.int32),
            pltpu.VMEM((CHUNK + LANES,), jnp.int32),
            pltpu.VMEM((CHUNK, 2 * D), jnp.float32),
            pltpu.VMEM((CHUNK // 2, 2 * D), jnp.float32),
            pltpu.VMEM((S, 2 * D), jnp.float32),
            pltpu.SemaphoreType.DMA,
        ],
    )
    def k(idx2_hbm, pb_hbm, tab2_hbm, pos2_hbm, out_hbm,
          idx_v, pb_v, rows_v, res_v, pos_v, sem):
        wid = lax.axis_index("s") * NC + lax.axis_index("c")
        pltpu.sync_copy(pos2_hbm, pos_v)
        iota = lax.iota(jnp.int32, LANES)

        @pl.loop(0, chunks, init_carry=jnp.int32(0))
        def _chunk(c, pos0):
            pltpu.sync_copy(idx2_hbm.at[wid, c], idx_v)
            pltpu.sync_copy(pb_hbm.at[wid, c], pb_v.at[pl.ds(0, CHUNK)])
            pltpu.async_copy(tab2_hbm.at[idx_v], rows_v, sem).wait()

            @pl.loop(0, CHUNK, init_carry=pos0)
            def _row(r, posr):
                half = pb_v[pl.ds(r, LANES)][0]
                outh = (r & 1) * D
                for j in range(D // LANES):
                    g = rows_v[r, pl.ds(half + 16 * j, LANES)]
                    res = g + pos_v[posr, pl.ds(16 * j, LANES)]
                    res_v[r >> 1, pl.ds(outh + 16 * j, LANES)] = res
                posr = posr + 1
                return jnp.where(posr == S, jnp.int32(0), posr)

            pltpu.sync_copy(res_v,
                            out_hbm.at[pl.ds(wid * (rows_per_w // 2)
                                             + c * (CHUNK // 2), CHUNK // 2)])
            return _row

    return k


def kernel(x, token_table, pos_table):
    B, S = x.shape
    V, D = token_table.shape
    xf = x.reshape(-1).astype(jnp.int32)
    rows_total = B * S
    chunks = rows_total // NW // CHUNK
    idx2 = (xf >> 1).reshape(NW, chunks, CHUNK)
    pb = ((xf & 1) * D).reshape(NW, chunks, CHUNK)
    tab2 = token_table.reshape(V // 2, 2 * D)
    pos2 = jnp.pad(pos_table, ((0, 0), (0, D)))
    k = _make_kernel(B, S, V, D)
    out = k(idx2, pb, tab2, pos2)
    return out.reshape(B, S, D)


# speedup vs baseline: 1.0777x; 1.0777x over previous
import functools

import jax
import jax.numpy as jnp
from jax import lax
from jax.experimental import pallas as pl
from jax.experimental.pallas import tpu as pltpu
from jax.experimental.pallas import tpu_sc as plsc

NC = 2
NS = 16
NW = NC * NS
LANES = 16
CHUNK = 128


def _make_kernel(B, S, V, D):
    rows_total = B * S
    rows_per_w = rows_total // NW
    chunks = rows_per_w // CHUNK
    assert rows_per_w % CHUNK == 0 and rows_per_w % S == 0
    assert D == 64 and V % 2 == 0 and chunks % 2 == 0

    mesh = plsc.VectorSubcoreMesh(
        core_axis_name="c", subcore_axis_name="s",
        num_cores=NC, num_subcores=NS)

    @functools.partial(
        pl.kernel,
        out_type=jax.ShapeDtypeStruct((rows_total // 2, 2 * D), jnp.float32),
        mesh=mesh,
        scratch_types=[
            pltpu.VMEM((2, CHUNK), jnp.int32),
            pltpu.VMEM((2, CHUNK, LANES), jnp.int32),
            pltpu.VMEM((2, CHUNK, 2 * D), jnp.float32),
            pltpu.VMEM((2, CHUNK // 2, 2 * D), jnp.float32),
            pltpu.VMEM((S, 2 * D), jnp.float32),
            pltpu.SemaphoreType.DMA,
            pltpu.SemaphoreType.DMA,
            pltpu.SemaphoreType.DMA,
            pltpu.SemaphoreType.DMA,
        ],
    )
    def k(idx2_hbm, msk_hbm, tab2_hbm, pos2_hbm, out_hbm,
          idx_v, msk_v, rows_v, res_v, pos_v, gsem0, gsem1, osem0, osem1):
        gsems = (gsem0, gsem1)
        osems = (osem0, osem1)
        wid = lax.axis_index("s") * NC + lax.axis_index("c")
        pltpu.sync_copy(pos2_hbm, pos_v)

        def start_gather(c, b):
            pltpu.sync_copy(idx2_hbm.at[wid, c], idx_v.at[b])
            pltpu.sync_copy(msk_hbm.at[wid, c], msk_v.at[b])
            pltpu.async_copy(tab2_hbm.at[idx_v.at[b]], rows_v.at[b], gsems[b])

        def wait_gather(b):
            pltpu.make_async_copy(
                tab2_hbm.at[idx_v.at[b]], rows_v.at[b], gsems[b]).wait()

        def out_slice(c):
            return out_hbm.at[pl.ds(wid * (rows_per_w // 2)
                                    + c * (CHUNK // 2), CHUNK // 2)]

        start_gather(0, 0)

        @pl.loop(0, chunks // 2, init_carry=jnp.int32(0))
        def _pair(cc, pos0):
            posr = pos0
            for b in range(2):
                c = 2 * cc + b

                @pl.when(c + 1 < chunks)
                def _():
                    start_gather(c + 1, 1 - b)

                wait_gather(b)

                @pl.when(cc > 0)
                def _():
                    pltpu.make_async_copy(res_v.at[b], out_slice(c),
                                          osems[b]).wait()

                @pl.loop(0, CHUNK, init_carry=posr)
                def _row(r, posr):
                    m = msk_v[b, r, pl.ds(0, LANES)]
                    outh = (r & 1) * D
                    for j in range(D // LANES):
                        g0 = rows_v[b, r, pl.ds(16 * j, LANES)]
                        g1 = rows_v[b, r, pl.ds(D + 16 * j, LANES)]
                        g = jnp.where(m > 0, g1, g0)
                        res = g + pos_v[posr, pl.ds(16 * j, LANES)]
                        res_v[b, r >> 1, pl.ds(outh + 16 * j, LANES)] = res
                    posr = posr + 1
                    return jnp.where(posr == S, jnp.int32(0), posr)

                posr = _row
                pltpu.async_copy(res_v.at[b], out_slice(c), osems[b])
            return posr

        for b in range(2):
            pltpu.make_async_copy(res_v.at[b], out_slice(chunks - 2 + b),
                                  osems[b]).wait()

    return k


def kernel(x, token_table, pos_table):
    B, S = x.shape
    V, D = token_table.shape
    xf = x.reshape(-1).astype(jnp.int32)
    rows_total = B * S
    chunks = rows_total // NW // CHUNK
    idx2 = (xf >> 1).reshape(NW, chunks, CHUNK)
    msk = jnp.broadcast_to((xf & 1).reshape(NW, chunks, CHUNK, 1),
                           (NW, chunks, CHUNK, LANES))
    tab2 = token_table.reshape(V // 2, 2 * D)
    pos2 = jnp.pad(pos_table, ((0, 0), (0, D)))
    k = _make_kernel(B, S, V, D)
    out = k(idx2, msk, tab2, pos2)
    return out.reshape(B, S, D)


# speedup vs baseline: 1.1829x; 1.0977x over previous
import functools

import jax
import jax.numpy as jnp
from jax import lax
from jax.experimental import pallas as pl
from jax.experimental.pallas import tpu as pltpu
from jax.experimental.pallas import tpu_sc as plsc

NC = 2
NS = 16
NW = NC * NS
LANES = 16
CHUNK = 128


TB = 1024


def _repack_split(V):
    nblk = -(-V // (2 * TB))
    return nblk * TB, nblk


def _make_repack(V, D):
    hsplit, nblk = _repack_split(V)
    last = -(-V // TB) - 1

    def body(a_ref, b_ref, out_ref):
        out_ref[:, 0:D] = jnp.transpose(a_ref[...], (1, 0))
        out_ref[:, D:2 * D] = jnp.transpose(b_ref[...], (1, 0))

    return pl.pallas_call(
        body,
        grid=(nblk,),
        in_specs=[
            pl.BlockSpec((D, TB), lambda i: (0, i)),
            pl.BlockSpec((D, TB), lambda i: (0, jnp.minimum(i + nblk, last))),
        ],
        out_specs=pl.BlockSpec((TB, 2 * D), lambda i: (i, 0)),
        out_shape=jax.ShapeDtypeStruct((hsplit, 2 * D), jnp.float32),
    )


def _make_kernel(B, S, V, D):
    rows_total = B * S
    rows_per_w = rows_total // NW
    chunks = rows_per_w // CHUNK
    assert rows_per_w % CHUNK == 0 and rows_per_w % S == 0
    assert D == 64 and V % 2 == 0 and chunks % 2 == 0

    mesh = plsc.VectorSubcoreMesh(
        core_axis_name="c", subcore_axis_name="s",
        num_cores=NC, num_subcores=NS)

    @functools.partial(
        pl.kernel,
        out_type=jax.ShapeDtypeStruct((rows_total // 2, 2 * D), jnp.float32),
        mesh=mesh,
        scratch_types=[
            pltpu.VMEM((2, CHUNK), jnp.int32),
            pltpu.VMEM((2, CHUNK, LANES), jnp.int32),
            pltpu.VMEM((2, CHUNK, 2 * D), jnp.float32),
            pltpu.VMEM((2, CHUNK // 2, 2 * D), jnp.float32),
            pltpu.VMEM((2, CHUNK // 2, 2 * D), jnp.float32),
            pltpu.SemaphoreType.DMA,
            pltpu.SemaphoreType.DMA,
            pltpu.SemaphoreType.DMA,
            pltpu.SemaphoreType.DMA,
        ],
    )
    def k(idx2_hbm, msk_hbm, tab2_hbm, post_hbm, out_hbm,
          idx_v, msk_v, rows_v, res_v, pos_v, gsem0, gsem1, osem0, osem1):
        gsems = (gsem0, gsem1)
        osems = (osem0, osem1)
        wid = lax.axis_index("s") * NC + lax.axis_index("c")

        def start_gather(c, b):
            pltpu.sync_copy(idx2_hbm.at[wid, c], idx_v.at[b])
            pltpu.sync_copy(msk_hbm.at[wid, c], msk_v.at[b])
            pltpu.sync_copy(post_hbm.at[c], pos_v.at[b])
            pltpu.async_copy(tab2_hbm.at[idx_v.at[b]], rows_v.at[b], gsems[b])

        def wait_gather(b):
            pltpu.make_async_copy(
                tab2_hbm.at[idx_v.at[b]], rows_v.at[b], gsems[b]).wait()

        def out_slice(c):
            return out_hbm.at[pl.ds(wid * (rows_per_w // 2)
                                    + c * (CHUNK // 2), CHUNK // 2)]

        start_gather(0, 0)

        @pl.loop(0, chunks // 2)
        def _pair(cc):
            for b in range(2):
                c = 2 * cc + b

                @pl.when(c + 1 < chunks)
                def _():
                    start_gather(c + 1, 1 - b)

                wait_gather(b)

                @pl.when(cc > 0)
                def _():
                    pltpu.make_async_copy(res_v.at[b], out_slice(c),
                                          osems[b]).wait()

                @pl.loop(0, CHUNK // 2, unroll=2)
                def _orow(q):
                    r = 2 * q
                    for p in range(2):
                        m = msk_v[b, r + p, pl.ds(0, LANES)]
                        for j in range(D // LANES):
                            g0 = rows_v[b, r + p, pl.ds(16 * j, LANES)]
                            g1 = rows_v[b, r + p, pl.ds(D + 16 * j, LANES)]
                            g = jnp.where(m > 0, g1, g0)
                            res = g + pos_v[b, q, pl.ds(p * D + 16 * j, LANES)]
                            res_v[b, q, pl.ds(p * D + 16 * j, LANES)] = res

                pltpu.async_copy(res_v.at[b], out_slice(c), osems[b])

        for b in range(2):
            pltpu.make_async_copy(res_v.at[b], out_slice(chunks - 2 + b),
                                  osems[b]).wait()

    return k


def kernel(x, token_table, pos_table):
    B, S = x.shape
    V, D = token_table.shape
    xf = x.reshape(-1).astype(jnp.int32)
    rows_total = B * S
    chunks = rows_total // NW // CHUNK
    V2, _ = _repack_split(V)
    hi = xf >= V2
    idx2 = jnp.where(hi, xf - V2, xf).reshape(NW, chunks, CHUNK)
    msk = jnp.broadcast_to(hi.astype(jnp.int32).reshape(NW, chunks, CHUNK, 1),
                           (NW, chunks, CHUNK, LANES))
    tabT = token_table.T
    tab2 = _make_repack(V, D)(tabT, tabT)
    period = (S * CHUNK) // _gcd(S * CHUNK, CHUNK * CHUNK)
    reps = S // _gcd(S, CHUNK)
    tile0 = jnp.tile(pos_table, (CHUNK * reps // S, 1))
    tile0 = tile0.reshape(reps, CHUNK // 2, 2 * D)
    post = jnp.tile(tile0, (chunks // reps, 1, 1))
    k = _make_kernel(B, S, V, D)
    out = k(idx2, msk, tab2, post)
    return out.reshape(B, S, D)


def _gcd(a, b):
    while b:
        a, b = b, a % b
    return a


# speedup vs baseline: 1.4980x; 1.2663x over previous
import functools

import jax
import jax.numpy as jnp
from jax import lax
from jax.experimental import pallas as pl
from jax.experimental.pallas import tpu as pltpu
from jax.experimental.pallas import tpu_sc as plsc

NC = 2
NS = 16
NW = NC * NS
LANES = 16
CHUNK = 128


TB = 1024


def _repack_split(V):
    nblk = -(-V // (2 * TB))
    return nblk * TB, nblk


def _make_repack(V, D):
    hsplit, nblk = _repack_split(V)
    last = -(-V // TB) - 1

    def body(a_ref, b_ref, out_ref):
        out_ref[:, 0:D] = jnp.transpose(a_ref[...], (1, 0))
        out_ref[:, D:2 * D] = jnp.transpose(b_ref[...], (1, 0))

    return pl.pallas_call(
        body,
        grid=(nblk,),
        in_specs=[
            pl.BlockSpec((D, TB), lambda i: (0, i)),
            pl.BlockSpec((D, TB), lambda i: (0, jnp.minimum(i + nblk, last))),
        ],
        out_specs=pl.BlockSpec((TB, 2 * D), lambda i: (i, 0)),
        out_shape=jax.ShapeDtypeStruct((hsplit, 2 * D), jnp.float32),
    )


def _make_kernel(B, S, V, D):
    rows_total = B * S
    rows_per_w = rows_total // NW
    chunks = rows_per_w // CHUNK
    assert rows_per_w % CHUNK == 0 and rows_per_w % S == 0
    assert D == 64 and V % 2 == 0 and chunks % 2 == 0

    mesh = plsc.VectorSubcoreMesh(
        core_axis_name="c", subcore_axis_name="s",
        num_cores=NC, num_subcores=NS)

    @functools.partial(
        pl.kernel,
        out_type=jax.ShapeDtypeStruct((rows_total // 8, 8, D), jnp.float32),
        mesh=mesh,
        scratch_types=[
            pltpu.VMEM((2, CHUNK), jnp.int32),
            pltpu.VMEM((2, CHUNK, LANES), jnp.int32),
            pltpu.VMEM((2, CHUNK, 2 * D), jnp.float32),
            pltpu.VMEM((2, CHUNK // 8, 8, D), jnp.float32),
            pltpu.SemaphoreType.DMA,
            pltpu.SemaphoreType.DMA,
            pltpu.SemaphoreType.DMA,
            pltpu.SemaphoreType.DMA,
            pltpu.SemaphoreType.DMA,
            pltpu.SemaphoreType.DMA,
        ],
    )
    def k(idx2_hbm, msk_hbm, tab2_hbm, post_hbm, out_hbm,
          idx_v, msk_v, rows_v, res_v,
          gsem0, gsem1, osem0, osem1, xsem0, xsem1):
        gsems = (gsem0, gsem1)
        osems = (osem0, osem1)
        xsems = (xsem0, xsem1)
        wid = lax.axis_index("s") * NC + lax.axis_index("c")

        def start_gather(c, b):
            pltpu.sync_copy(idx2_hbm.at[wid, c], idx_v.at[b])
            pltpu.async_copy(tab2_hbm.at[idx_v.at[b]], rows_v.at[b], gsems[b])
            pltpu.async_copy(msk_hbm.at[wid, c], msk_v.at[b], xsems[b])
            pltpu.async_copy(post_hbm.at[c], res_v.at[b], xsems[b])

        def wait_gather(c, b):
            pltpu.make_async_copy(
                tab2_hbm.at[idx_v.at[b]], rows_v.at[b], gsems[b]).wait()
            pltpu.make_async_copy(
                msk_hbm.at[wid, c], msk_v.at[b], xsems[b]).wait()
            pltpu.make_async_copy(
                post_hbm.at[c], res_v.at[b], xsems[b]).wait()

        def out_slice(c):
            return out_hbm.at[pl.ds(wid * (rows_per_w // 8)
                                    + c * (CHUNK // 8), CHUNK // 8)]

        start_gather(0, 0)

        @pl.loop(0, chunks // 2)
        def _pair(cc):
            for b in range(2):
                c = 2 * cc + b

                @pl.when(c + 1 < chunks)
                def _():
                    @pl.when(c >= 1)
                    def _():
                        pltpu.make_async_copy(res_v.at[1 - b],
                                              out_slice(c - 1),
                                              osems[1 - b]).wait()
                    start_gather(c + 1, 1 - b)

                wait_gather(c, b)

                @pl.loop(0, CHUNK // 8, unroll=2)
                def _oblk(q):
                    for p in range(8):
                        r = 8 * q + p
                        m = msk_v[b, r, pl.ds(0, LANES)]
                        for j in range(D // LANES):
                            g0 = rows_v[b, r, pl.ds(16 * j, LANES)]
                            g1 = rows_v[b, r, pl.ds(D + 16 * j, LANES)]
                            g = jnp.where(m > 0, g1, g0)
                            res = g + res_v[b, q, p, pl.ds(16 * j, LANES)]
                            res_v[b, q, p, pl.ds(16 * j, LANES)] = res

                pltpu.async_copy(res_v.at[b], out_slice(c), osems[b])

        for b in range(2):
            pltpu.make_async_copy(res_v.at[b], out_slice(chunks - 2 + b),
                                  osems[b]).wait()

    return k


def kernel(x, token_table, pos_table):
    B, S = x.shape
    V, D = token_table.shape
    xf = x.reshape(-1).astype(jnp.int32)
    rows_total = B * S
    chunks = rows_total // NW // CHUNK
    V2, _ = _repack_split(V)
    hi = xf >= V2
    idx2 = jnp.where(hi, xf - V2, xf).reshape(NW, chunks, CHUNK)
    msk = jnp.broadcast_to(hi.astype(jnp.int32).reshape(NW, chunks, CHUNK, 1),
                           (NW, chunks, CHUNK, LANES))
    tabT = token_table.T
    tab2 = _make_repack(V, D)(tabT, tabT)
    reps = S // _gcd(S, CHUNK)
    tile0 = jnp.tile(pos_table, (CHUNK * reps // S, 1))
    tile0 = tile0.reshape(reps, CHUNK // 8, 8, D)
    post = jnp.tile(tile0, (chunks // reps, 1, 1, 1))
    k = _make_kernel(B, S, V, D)
    out = k(idx2, msk, tab2, post)
    return out.reshape(B, S, D)


def _gcd(a, b):
    while b:
        a, b = b, a % b
    return a


# speedup vs baseline: 1.8001x; 1.2017x over previous
import functools

import jax
import jax.numpy as jnp
from jax import lax
from jax.experimental import pallas as pl
from jax.experimental.pallas import tpu as pltpu
from jax.experimental.pallas import tpu_sc as plsc

NC = 2
NS = 16
NW = NC * NS
LANES = 16
CHUNK = 128


TB = 2048


def _repack_split(V):
    nblk = -(-V // (2 * TB))
    return nblk * TB, nblk


def _make_repack(V, D):
    hsplit, nblk = _repack_split(V)
    last = -(-V // TB) - 1

    def body(a_ref, b_ref, out_ref):
        out_ref[...] = jnp.concatenate(
            [jnp.transpose(a_ref[...], (1, 0)),
             jnp.transpose(b_ref[...], (1, 0))], axis=1)

    return pl.pallas_call(
        body,
        grid=(nblk,),
        in_specs=[
            pl.BlockSpec((D, TB), lambda i: (0, i)),
            pl.BlockSpec((D, TB), lambda i: (0, jnp.minimum(i + nblk, last))),
        ],
        out_specs=pl.BlockSpec((TB, 2 * D), lambda i: (i, 0)),
        out_shape=jax.ShapeDtypeStruct((hsplit, 2 * D), jnp.float32),
    )


def _make_kernel(B, S, V, D):
    rows_total = B * S
    rows_per_w = rows_total // NW
    chunks = rows_per_w // CHUNK
    assert rows_per_w % CHUNK == 0 and rows_per_w % S == 0
    assert D == 64 and V % 2 == 0 and chunks % 2 == 0

    mesh = plsc.VectorSubcoreMesh(
        core_axis_name="c", subcore_axis_name="s",
        num_cores=NC, num_subcores=NS)

    @functools.partial(
        pl.kernel,
        out_type=jax.ShapeDtypeStruct((rows_total // 8, 8, D), jnp.float32),
        mesh=mesh,
        scratch_types=[
            pltpu.VMEM((2, CHUNK), jnp.int32),
            pltpu.VMEM((2, CHUNK, LANES), jnp.int32),
            pltpu.VMEM((2, CHUNK, 2 * D), jnp.float32),
            pltpu.VMEM((2, CHUNK // 8, 8, D), jnp.float32),
            pltpu.SemaphoreType.DMA,
            pltpu.SemaphoreType.DMA,
            pltpu.SemaphoreType.DMA,
            pltpu.SemaphoreType.DMA,
            pltpu.SemaphoreType.DMA,
            pltpu.SemaphoreType.DMA,
        ],
    )
    def k(idx2_hbm, msk_hbm, tab2_hbm, post_hbm, out_hbm,
          idx_v, msk_v, rows_v, res_v,
          gsem0, gsem1, osem0, osem1, xsem0, xsem1):
        gsems = (gsem0, gsem1)
        osems = (osem0, osem1)
        xsems = (xsem0, xsem1)
        wid = lax.axis_index("s") * NC + lax.axis_index("c")

        def start_gather(c, b):
            pltpu.sync_copy(idx2_hbm.at[wid, c], idx_v.at[b])
            pltpu.async_copy(tab2_hbm.at[idx_v.at[b]], rows_v.at[b], gsems[b])
            pltpu.async_copy(msk_hbm.at[wid, c], msk_v.at[b], xsems[b])
            pltpu.async_copy(post_hbm.at[c], res_v.at[b], xsems[b])

        def wait_gather(c, b):
            pltpu.make_async_copy(
                tab2_hbm.at[idx_v.at[b]], rows_v.at[b], gsems[b]).wait()
            pltpu.make_async_copy(
                msk_hbm.at[wid, c], msk_v.at[b], xsems[b]).wait()
            pltpu.make_async_copy(
                post_hbm.at[c], res_v.at[b], xsems[b]).wait()

        def out_slice(c):
            return out_hbm.at[pl.ds(wid * (rows_per_w // 8)
                                    + c * (CHUNK // 8), CHUNK // 8)]

        start_gather(0, 0)

        @pl.loop(0, chunks // 2)
        def _pair(cc):
            for b in range(2):
                c = 2 * cc + b

                @pl.when(c + 1 < chunks)
                def _():
                    @pl.when(c >= 1)
                    def _():
                        pltpu.make_async_copy(res_v.at[1 - b],
                                              out_slice(c - 1),
                                              osems[1 - b]).wait()
                    start_gather(c + 1, 1 - b)

                wait_gather(c, b)

                @pl.loop(0, CHUNK // 8, unroll=2)
                def _oblk(q):
                    for p in range(8):
                        r = 8 * q + p
                        m = msk_v[b, r, pl.ds(0, LANES)]
                        for j in range(D // LANES):
                            g0 = rows_v[b, r, pl.ds(16 * j, LANES)]
                            g1 = rows_v[b, r, pl.ds(D + 16 * j, LANES)]
                            g = jnp.where(m > 0, g1, g0)
                            res = g + res_v[b, q, p, pl.ds(16 * j, LANES)]
                            res_v[b, q, p, pl.ds(16 * j, LANES)] = res

                pltpu.async_copy(res_v.at[b], out_slice(c), osems[b])

        for b in range(2):
            pltpu.make_async_copy(res_v.at[b], out_slice(chunks - 2 + b),
                                  osems[b]).wait()

    return k


def kernel(x, token_table, pos_table):
    B, S = x.shape
    V, D = token_table.shape
    xf = x.reshape(-1).astype(jnp.int32)
    rows_total = B * S
    chunks = rows_total // NW // CHUNK
    V2, _ = _repack_split(V)
    hi = xf >= V2
    idx2 = jnp.where(hi, xf - V2, xf).reshape(NW, chunks, CHUNK)
    msk = jnp.broadcast_to(hi.astype(jnp.int32).reshape(NW, chunks, CHUNK, 1),
                           (NW, chunks, CHUNK, LANES))
    tabT = token_table.T
    tab2 = _make_repack(V, D)(tabT, tabT)
    reps = S // _gcd(S, CHUNK)
    tile0 = jnp.tile(pos_table, (CHUNK * reps // S, 1))
    tile0 = tile0.reshape(reps, CHUNK // 8, 8, D)
    post = jnp.tile(tile0, (chunks // reps, 1, 1, 1))
    k = _make_kernel(B, S, V, D)
    out = k(idx2, msk, tab2, post)
    return out.reshape(B, S, D)


def _gcd(a, b):
    while b:
        a, b = b, a % b
    return a


# speedup vs baseline: 2.0166x; 1.1203x over previous
import functools

import jax
import jax.numpy as jnp
from jax import lax
from jax.experimental import pallas as pl
from jax.experimental.pallas import tpu as pltpu
from jax.experimental.pallas import tpu_sc as plsc

NC = 2
NS = 16
NW = NC * NS
LANES = 16
CHUNK = 128


TB = 4096


def _repack_split(V):
    nblk = -(-V // (2 * TB))
    return nblk * TB, nblk


def _make_repack(V, D):
    hsplit, nblk = _repack_split(V)
    last = -(-V // TB) - 1

    def body(a_ref, b_ref, out_ref):
        out_ref[...] = jnp.concatenate(
            [jnp.transpose(a_ref[...], (1, 0)),
             jnp.transpose(b_ref[...], (1, 0))], axis=1)

    return pl.pallas_call(
        body,
        grid=(nblk,),
        in_specs=[
            pl.BlockSpec((D, TB), lambda i: (0, i)),
            pl.BlockSpec((D, TB), lambda i: (0, jnp.minimum(i + nblk, last))),
        ],
        out_specs=pl.BlockSpec((TB, 2 * D), lambda i: (i, 0)),
        out_shape=jax.ShapeDtypeStruct((hsplit, 2 * D), jnp.float32),
    )


def _make_kernel(B, S, V, D):
    rows_total = B * S
    rows_per_w = rows_total // NW
    chunks = rows_per_w // CHUNK
    assert rows_per_w % CHUNK == 0 and rows_per_w % S == 0
    assert D == 64 and V % 2 == 0 and chunks % 2 == 0

    mesh = plsc.VectorSubcoreMesh(
        core_axis_name="c", subcore_axis_name="s",
        num_cores=NC, num_subcores=NS)

    @functools.partial(
        pl.kernel,
        out_type=jax.ShapeDtypeStruct((rows_total // 8, 8, D), jnp.float32),
        mesh=mesh,
        scratch_types=[
            pltpu.VMEM((2, CHUNK), jnp.int32),
            pltpu.VMEM((2, CHUNK, LANES), jnp.int32),
            pltpu.VMEM((2, CHUNK, 2 * D), jnp.float32),
            pltpu.VMEM((2, CHUNK // 8, 8, D), jnp.float32),
            pltpu.SemaphoreType.DMA,
            pltpu.SemaphoreType.DMA,
            pltpu.SemaphoreType.DMA,
            pltpu.SemaphoreType.DMA,
            pltpu.SemaphoreType.DMA,
            pltpu.SemaphoreType.DMA,
        ],
    )
    def k(idx2_hbm, msk_hbm, tab2_hbm, post_hbm, out_hbm,
          idx_v, msk_v, rows_v, res_v,
          gsem0, gsem1, osem0, osem1, xsem0, xsem1):
        gsems = (gsem0, gsem1)
        osems = (osem0, osem1)
        xsems = (xsem0, xsem1)
        wid = lax.axis_index("s") * NC + lax.axis_index("c")

        def start_gather(c, b):
            pltpu.sync_copy(idx2_hbm.at[wid, c], idx_v.at[b])
            pltpu.async_copy(tab2_hbm.at[idx_v.at[b]], rows_v.at[b], gsems[b])
            pltpu.async_copy(msk_hbm.at[wid, c], msk_v.at[b], xsems[b])
            pltpu.async_copy(post_hbm.at[c], res_v.at[b], xsems[b])

        def wait_gather(c, b):
            pltpu.make_async_copy(
                tab2_hbm.at[idx_v.at[b]], rows_v.at[b], gsems[b]).wait()
            pltpu.make_async_copy(
                msk_hbm.at[wid, c], msk_v.at[b], xsems[b]).wait()
            pltpu.make_async_copy(
                post_hbm.at[c], res_v.at[b], xsems[b]).wait()

        def out_slice(c):
            return out_hbm.at[pl.ds(wid * (rows_per_w // 8)
                                    + c * (CHUNK // 8), CHUNK // 8)]

        start_gather(0, 0)

        @pl.loop(0, chunks // 2)
        def _pair(cc):
            for b in range(2):
                c = 2 * cc + b

                @pl.when(c + 1 < chunks)
                def _():
                    @pl.when(c >= 1)
                    def _():
                        pltpu.make_async_copy(res_v.at[1 - b],
                                              out_slice(c - 1),
                                              osems[1 - b]).wait()
                    start_gather(c + 1, 1 - b)

                wait_gather(c, b)

                @pl.loop(0, CHUNK // 8, unroll=4)
                def _oblk(q):
                    for p in range(8):
                        r = 8 * q + p
                        m = msk_v[b, r, pl.ds(0, LANES)]
                        for j in range(D // LANES):
                            g0 = rows_v[b, r, pl.ds(16 * j, LANES)]
                            g1 = rows_v[b, r, pl.ds(D + 16 * j, LANES)]
                            g = jnp.where(m > 0, g1, g0)
                            res = g + res_v[b, q, p, pl.ds(16 * j, LANES)]
                            res_v[b, q, p, pl.ds(16 * j, LANES)] = res

                pltpu.async_copy(res_v.at[b], out_slice(c), osems[b])

        for b in range(2):
            pltpu.make_async_copy(res_v.at[b], out_slice(chunks - 2 + b),
                                  osems[b]).wait()

    return k


def kernel(x, token_table, pos_table):
    B, S = x.shape
    V, D = token_table.shape
    xf = x.reshape(-1).astype(jnp.int32)
    rows_total = B * S
    chunks = rows_total // NW // CHUNK
    V2, _ = _repack_split(V)
    hi = xf >= V2
    idx2 = jnp.where(hi, xf - V2, xf).reshape(NW, chunks, CHUNK)
    msk = jnp.broadcast_to(hi.astype(jnp.int32).reshape(NW, chunks, CHUNK, 1),
                           (NW, chunks, CHUNK, LANES))
    tabT = token_table.T
    tab2 = _make_repack(V, D)(tabT, tabT)
    reps = S // _gcd(S, CHUNK)
    tile0 = jnp.tile(pos_table, (CHUNK * reps // S, 1))
    tile0 = tile0.reshape(reps, CHUNK // 8, 8, D)
    post = jnp.tile(tile0, (chunks // reps, 1, 1, 1))
    k = _make_kernel(B, S, V, D)
    out = k(idx2, msk, tab2, post)
    return out.reshape(B, S, D)


def _gcd(a, b):
    while b:
        a, b = b, a % b
    return a


# speedup vs baseline: 2.1501x; 1.0662x over previous
import functools

import jax
import jax.numpy as jnp
from jax import lax
from jax.experimental import pallas as pl
from jax.experimental.pallas import tpu as pltpu
from jax.experimental.pallas import tpu_sc as plsc

NC = 2
NS = 16
NW = NC * NS
LANES = 16
CHUNK = 128


TB = 8192


def _repack_split(V):
    nblk = -(-V // (2 * TB))
    return nblk * TB, nblk


def _make_repack(V, D):
    hsplit, nblk = _repack_split(V)
    last = -(-V // TB) - 1

    def body(a_ref, b_ref, out_ref):
        out_ref[...] = jnp.concatenate(
            [jnp.transpose(a_ref[...], (1, 0)),
             jnp.transpose(b_ref[...], (1, 0))], axis=1)

    return pl.pallas_call(
        body,
        grid=(nblk,),
        in_specs=[
            pl.BlockSpec((D, TB), lambda i: (0, i)),
            pl.BlockSpec((D, TB), lambda i: (0, jnp.minimum(i + nblk, last))),
        ],
        out_specs=pl.BlockSpec((TB, 2 * D), lambda i: (i, 0)),
        out_shape=jax.ShapeDtypeStruct((hsplit, 2 * D), jnp.float32),
    )


def _make_kernel(B, S, V, D):
    rows_total = B * S
    rows_per_w = rows_total // NW
    chunks = rows_per_w // CHUNK
    assert rows_per_w % CHUNK == 0 and rows_per_w % S == 0
    assert D == 64 and V % 2 == 0 and chunks % 2 == 0

    mesh = plsc.VectorSubcoreMesh(
        core_axis_name="c", subcore_axis_name="s",
        num_cores=NC, num_subcores=NS)

    @functools.partial(
        pl.kernel,
        out_type=jax.ShapeDtypeStruct((rows_total // 8, 8, D), jnp.float32),
        mesh=mesh,
        scratch_types=[
            pltpu.VMEM((2, CHUNK), jnp.int32),
            pltpu.VMEM((2, CHUNK, LANES), jnp.int32),
            pltpu.VMEM((2, CHUNK, 2 * D), jnp.float32),
            pltpu.VMEM((2, CHUNK // 8, 8, D), jnp.float32),
            pltpu.SemaphoreType.DMA,
            pltpu.SemaphoreType.DMA,
            pltpu.SemaphoreType.DMA,
            pltpu.SemaphoreType.DMA,
            pltpu.SemaphoreType.DMA,
            pltpu.SemaphoreType.DMA,
        ],
    )
    def k(idx2_hbm, msk_hbm, tab2_hbm, post_hbm, out_hbm,
          idx_v, msk_v, rows_v, res_v,
          gsem0, gsem1, osem0, osem1, xsem0, xsem1):
        gsems = (gsem0, gsem1)
        osems = (osem0, osem1)
        xsems = (xsem0, xsem1)
        wid = lax.axis_index("s") * NC + lax.axis_index("c")

        def start_gather(c, b):
            pltpu.sync_copy(idx2_hbm.at[wid, c], idx_v.at[b])
            pltpu.async_copy(tab2_hbm.at[idx_v.at[b]], rows_v.at[b], gsems[b])
            pltpu.async_copy(msk_hbm.at[wid, c], msk_v.at[b], xsems[b])
            pltpu.async_copy(post_hbm.at[c], res_v.at[b], xsems[b])

        def wait_gather(c, b):
            pltpu.make_async_copy(
                tab2_hbm.at[idx_v.at[b]], rows_v.at[b], gsems[b]).wait()
            pltpu.make_async_copy(
                msk_hbm.at[wid, c], msk_v.at[b], xsems[b]).wait()
            pltpu.make_async_copy(
                post_hbm.at[c], res_v.at[b], xsems[b]).wait()

        def out_slice(c):
            return out_hbm.at[pl.ds(wid * (rows_per_w // 8)
                                    + c * (CHUNK // 8), CHUNK // 8)]

        start_gather(0, 0)

        @pl.loop(0, chunks // 2)
        def _pair(cc):
            for b in range(2):
                c = 2 * cc + b

                @pl.when(c + 1 < chunks)
                def _():
                    @pl.when(c >= 1)
                    def _():
                        pltpu.make_async_copy(res_v.at[1 - b],
                                              out_slice(c - 1),
                                              osems[1 - b]).wait()
                    start_gather(c + 1, 1 - b)

                wait_gather(c, b)

                @pl.loop(0, CHUNK // 8, unroll=4)
                def _oblk(q):
                    for p in range(8):
                        r = 8 * q + p
                        m = msk_v[b, r, pl.ds(0, LANES)]
                        for j in range(D // LANES):
                            g0 = rows_v[b, r, pl.ds(16 * j, LANES)]
                            g1 = rows_v[b, r, pl.ds(D + 16 * j, LANES)]
                            g = jnp.where(m > 0, g1, g0)
                            res = g + res_v[b, q, p, pl.ds(16 * j, LANES)]
                            res_v[b, q, p, pl.ds(16 * j, LANES)] = res

                pltpu.async_copy(res_v.at[b], out_slice(c), osems[b])

        for b in range(2):
            pltpu.make_async_copy(res_v.at[b], out_slice(chunks - 2 + b),
                                  osems[b]).wait()

    return k


def kernel(x, token_table, pos_table):
    B, S = x.shape
    V, D = token_table.shape
    xf = x.reshape(-1).astype(jnp.int32)
    rows_total = B * S
    chunks = rows_total // NW // CHUNK
    V2, _ = _repack_split(V)
    hi = xf >= V2
    idx2 = jnp.where(hi, xf - V2, xf).reshape(NW, chunks, CHUNK)
    msk = jnp.broadcast_to(hi.astype(jnp.int32).reshape(NW, chunks, CHUNK, 1),
                           (NW, chunks, CHUNK, LANES))
    tabT = token_table.T
    tab2 = _make_repack(V, D)(tabT, tabT)
    reps = S // _gcd(S, CHUNK)
    tile0 = jnp.tile(pos_table, (CHUNK * reps // S, 1))
    tile0 = tile0.reshape(reps, CHUNK // 8, 8, D)
    post = jnp.tile(tile0, (chunks // reps, 1, 1, 1))
    k = _make_kernel(B, S, V, D)
    out = k(idx2, msk, tab2, post)
    return out.reshape(B, S, D)


def _gcd(a, b):
    while b:
        a, b = b, a % b
    return a


# speedup vs baseline: 2.2056x; 1.0258x over previous
import functools

import jax
import jax.numpy as jnp
from jax import lax
from jax.experimental import pallas as pl
from jax.experimental.pallas import tpu as pltpu
from jax.experimental.pallas import tpu_sc as plsc

NC = 2
NS = 16
NW = NC * NS
LANES = 16
CHUNK = 128


TB = 16384


def _repack_split(V):
    nblk = -(-V // (2 * TB))
    return nblk * TB, nblk


def _make_repack(V, D):
    hsplit, nblk = _repack_split(V)
    last = -(-V // TB) - 1

    def body(a_ref, b_ref, out_ref):
        out_ref[...] = jnp.concatenate(
            [jnp.transpose(a_ref[...], (1, 0)),
             jnp.transpose(b_ref[...], (1, 0))], axis=1)

    return pl.pallas_call(
        body,
        grid=(nblk,),
        in_specs=[
            pl.BlockSpec((D, TB), lambda i: (0, i)),
            pl.BlockSpec((D, TB), lambda i: (0, jnp.minimum(i + nblk, last))),
        ],
        out_specs=pl.BlockSpec((TB, 2 * D), lambda i: (i, 0)),
        out_shape=jax.ShapeDtypeStruct((hsplit, 2 * D), jnp.float32),
    )


def _make_kernel(B, S, V, D):
    rows_total = B * S
    rows_per_w = rows_total // NW
    chunks = rows_per_w // CHUNK
    assert rows_per_w % CHUNK == 0 and rows_per_w % S == 0
    assert D == 64 and V % 2 == 0 and chunks % 2 == 0

    mesh = plsc.VectorSubcoreMesh(
        core_axis_name="c", subcore_axis_name="s",
        num_cores=NC, num_subcores=NS)

    @functools.partial(
        pl.kernel,
        out_type=jax.ShapeDtypeStruct((rows_total // 8, 8, D), jnp.float32),
        mesh=mesh,
        scratch_types=[
            pltpu.VMEM((2, CHUNK), jnp.int32),
            pltpu.VMEM((2, CHUNK, LANES), jnp.int32),
            pltpu.VMEM((2, CHUNK, 2 * D), jnp.float32),
            pltpu.VMEM((2, CHUNK // 8, 8, D), jnp.float32),
            pltpu.SemaphoreType.DMA,
            pltpu.SemaphoreType.DMA,
            pltpu.SemaphoreType.DMA,
            pltpu.SemaphoreType.DMA,
            pltpu.SemaphoreType.DMA,
            pltpu.SemaphoreType.DMA,
        ],
    )
    def k(idx2_hbm, msk_hbm, tab2_hbm, post_hbm, out_hbm,
          idx_v, msk_v, rows_v, res_v,
          gsem0, gsem1, osem0, osem1, xsem0, xsem1):
        gsems = (gsem0, gsem1)
        osems = (osem0, osem1)
        xsems = (xsem0, xsem1)
        wid = lax.axis_index("s") * NC + lax.axis_index("c")

        def start_gather(c, b):
            pltpu.sync_copy(idx2_hbm.at[wid, c], idx_v.at[b])
            pltpu.async_copy(tab2_hbm.at[idx_v.at[b]], rows_v.at[b], gsems[b])
            pltpu.async_copy(msk_hbm.at[wid, c], msk_v.at[b], xsems[b])
            pltpu.async_copy(post_hbm.at[c], res_v.at[b], xsems[b])

        def wait_gather(c, b):
            pltpu.make_async_copy(
                tab2_hbm.at[idx_v.at[b]], rows_v.at[b], gsems[b]).wait()
            pltpu.make_async_copy(
                msk_hbm.at[wid, c], msk_v.at[b], xsems[b]).wait()
            pltpu.make_async_copy(
                post_hbm.at[c], res_v.at[b], xsems[b]).wait()

        def out_slice(c):
            return out_hbm.at[pl.ds(wid * (rows_per_w // 8)
                                    + c * (CHUNK // 8), CHUNK // 8)]

        start_gather(0, 0)

        @pl.loop(0, chunks // 2)
        def _pair(cc):
            for b in range(2):
                c = 2 * cc + b

                @pl.when(c + 1 < chunks)
                def _():
                    @pl.when(c >= 1)
                    def _():
                        pltpu.make_async_copy(res_v.at[1 - b],
                                              out_slice(c - 1),
                                              osems[1 - b]).wait()
                    start_gather(c + 1, 1 - b)

                wait_gather(c, b)

                @pl.loop(0, CHUNK // 8, unroll=4)
                def _oblk(q):
                    for p in range(8):
                        r = 8 * q + p
                        m = msk_v[b, r, pl.ds(0, LANES)]
                        for j in range(D // LANES):
                            g0 = rows_v[b, r, pl.ds(16 * j, LANES)]
                            g1 = rows_v[b, r, pl.ds(D + 16 * j, LANES)]
                            g = jnp.where(m > 0, g1, g0)
                            res = g + res_v[b, q, p, pl.ds(16 * j, LANES)]
                            res_v[b, q, p, pl.ds(16 * j, LANES)] = res

                pltpu.async_copy(res_v.at[b], out_slice(c), osems[b])

        for b in range(2):
            pltpu.make_async_copy(res_v.at[b], out_slice(chunks - 2 + b),
                                  osems[b]).wait()

    return k


def kernel(x, token_table, pos_table):
    B, S = x.shape
    V, D = token_table.shape
    xf = x.reshape(-1).astype(jnp.int32)
    rows_total = B * S
    chunks = rows_total // NW // CHUNK
    V2, _ = _repack_split(V)
    hi = xf >= V2
    idx2 = jnp.where(hi, xf - V2, xf).reshape(NW, chunks, CHUNK)
    msk = jnp.broadcast_to(hi.astype(jnp.int32).reshape(NW, chunks, CHUNK, 1),
                           (NW, chunks, CHUNK, LANES))
    tabT = token_table.T
    tab2 = _make_repack(V, D)(tabT, tabT)
    reps = S // _gcd(S, CHUNK)
    tile0 = jnp.tile(pos_table, (CHUNK * reps // S, 1))
    tile0 = tile0.reshape(reps, CHUNK // 8, 8, D)
    post = jnp.tile(tile0, (chunks // reps, 1, 1, 1))
    k = _make_kernel(B, S, V, D)
    out = k(idx2, msk, tab2, post)
    return out.reshape(B, S, D)


def _gcd(a, b):
    while b:
        a, b = b, a % b
    return a
